# baseline (speedup 1.0000x reference)
import functools

import jax
import jax.numpy as jnp
from jax import lax
from jax.experimental import pallas as pl
from jax.experimental.pallas import tpu as pltpu
from jax.experimental.pallas import tpu_sc as plsc

_N = 10000
_E = 320000
_D = 128
_DE = 16
_EB = _E // 8

_NC = 2
_NS = 16
_NW = _NC * _NS
_EW = _E // _NW
_B = 1000
_NB = _EW // _B
_SB = 125
_JR = _B // _SB
_NP = 10240
_NPT = _NP // _NS


def _node_proj_body(x_ref, ws_ref, wr_ref, xs_ref, xr_ref):
    xv = x_ref[...]
    xs_ref[...] = jnp.dot(xv, ws_ref[...], preferred_element_type=jnp.float32)
    xr_ref[...] = jnp.dot(xv, wr_ref[...], preferred_element_type=jnp.float32)


def _edge_proj_body(ea_ref, bd_ref, b_ref, o_ref):
    o_ref[...] = jnp.dot(ea_ref[...], bd_ref[...],
                         preferred_element_type=jnp.float32) + b_ref[...]


def _node_mlp_body(x_ref, a0_ref, a1_ref, wt_ref, wb_ref, b_ref, o_ref):
    acc = jnp.dot(x_ref[...], wt_ref[...], preferred_element_type=jnp.float32)
    acc = acc + jnp.dot(a0_ref[...] + a1_ref[...], wb_ref[...],
                        preferred_element_type=jnp.float32)
    o_ref[...] = jnp.maximum(acc + b_ref[...], 0.0)


def _sc_edges_body(xs_hbm, xr_hbm, ea_hbm, s2_hbm, r2_hbm,
                   m_hbm, parts_hbm,
                   idx_s, idx_r, bufs, bufr, bufe, bufm, zbuf, agg_sh,
                   sem_g, sem_w):
    c = lax.axis_index("c")
    s = lax.axis_index("s")
    wid = c * _NS + s

    def _zero(i, carry):
        zbuf[i] = jnp.zeros((_DE,), jnp.float32)
        return carry
    lax.fori_loop(0, _NPT, _zero, 0)
    pltpu.sync_copy(zbuf, agg_sh.at[pl.ds(s * _NPT, _NPT)])
    plsc.subcore_barrier()

    e0 = wid * _EW

    def _block(b, carry):
        eb0 = e0 + b * _B
        pltpu.sync_copy(s2_hbm.at[pl.ds(eb0, _B)], idx_s)
        pltpu.sync_copy(r2_hbm.at[pl.ds(eb0, _B)], idx_r)
        pltpu.sync_copy(ea_hbm.at[pl.ds(eb0 * _DE, _B * _DE)], bufe)
        cps = [pltpu.async_copy(xs_hbm.at[idx_s], bufs, sem_g),
               pltpu.async_copy(xr_hbm.at[idx_r], bufr, sem_g)]
        for cp in cps:
            cp.wait()

        def _relu(i2, carry):
            for u in range(8):
                row = i2 * 8 + u
                val = jnp.maximum(
                    bufs[row] + bufr[row] + bufe[pl.ds(row * _DE, _DE)], 0.0)
                bufs[row] = val
                bufm[pl.ds(row * _DE, _DE)] = val
            return carry
        lax.fori_loop(0, _B // 8, _relu, 0)

        wcps = [pltpu.async_copy(bufm, m_hbm.at[pl.ds(eb0 * _DE, _B * _DE)],
                                 sem_w)]
        pltpu.sync_copy(bufs, agg_sh.at[idx_r], add=True)
        for cp in wcps:
            cp.wait()
        return carry

    lax.fori_loop(0, _NB, _block, 0)

    plsc.subcore_barrier()
    pltpu.sync_copy(agg_sh.at[pl.ds(s * _NPT, _NPT)], zbuf)
    pltpu.sync_copy(zbuf, parts_hbm.at[c, pl.ds(s * _NPT, _NPT)])


_sc_edges = functools.partial(
    pl.kernel,
    out_type=(jax.ShapeDtypeStruct((_E * _DE,), jnp.float32),
              jax.ShapeDtypeStruct((_NC, _NP, _DE), jnp.float32)),
    mesh=plsc.VectorSubcoreMesh(core_axis_name="c", subcore_axis_name="s"),
    compiler_params=pltpu.CompilerParams(use_tc_tiling_on_sc=False),
    scratch_types=[
        pltpu.VMEM((_B,), jnp.int32),
        pltpu.VMEM((_B,), jnp.int32),
        pltpu.VMEM((_B, _DE), jnp.float32),
        pltpu.VMEM((_B, _DE), jnp.float32),
        pltpu.VMEM((_B * _DE,), jnp.float32),
        pltpu.VMEM((_B * _DE,), jnp.float32),
        pltpu.VMEM((_NPT, _DE), jnp.float32),
        pltpu.VMEM_SHARED((_NP, _DE), jnp.float32),
        pltpu.SemaphoreType.DMA,
        pltpu.SemaphoreType.DMA,
    ],
)(_sc_edges_body)


@jax.jit
def kernel(x, edge_index, edge_attr, W_edge, b_edge, W_node, b_node):
    senders = edge_index[0]
    receivers = edge_index[1]
    w_s = W_edge[:_D]
    w_r = W_edge[_D:2 * _D]
    w_a = W_edge[2 * _D:]

    xs, xr = pl.pallas_call(
        _node_proj_body,
        out_shape=(jax.ShapeDtypeStruct((_N, _DE), jnp.float32),
                   jax.ShapeDtypeStruct((_N, _DE), jnp.float32)),
    )(x, w_s, w_r)

    ea2 = edge_attr.reshape(_EB, _D)
    bd = jnp.kron(jnp.eye(8, dtype=jnp.float32), w_a)
    bt = jnp.tile(b_edge, 8).reshape(1, _D)
    ea = pl.pallas_call(
        _edge_proj_body,
        grid=(10,),
        in_specs=[pl.BlockSpec((_EB // 10, _D), lambda i: (i, 0)),
                  pl.BlockSpec((_D, _D), lambda i: (0, 0)),
                  pl.BlockSpec((1, _D), lambda i: (0, 0))],
        out_specs=pl.BlockSpec((_EB // 10, _D), lambda i: (i, 0)),
        out_shape=jax.ShapeDtypeStruct((_EB, _D), jnp.float32),
    )(ea2, bd, bt)

    m_flat, parts = _sc_edges(xs, xr, ea.reshape(-1), senders, receivers)
    m = m_flat.reshape(_E, _DE)
    parts = parts[:, :_N]

    nodes = pl.pallas_call(
        _node_mlp_body,
        grid=(10,),
        in_specs=[pl.BlockSpec((_N // 10, _D), lambda i: (i, 0)),
                  pl.BlockSpec((_N // 10, _DE), lambda i: (i, 0)),
                  pl.BlockSpec((_N // 10, _DE), lambda i: (i, 0)),
                  pl.BlockSpec((_D, _D), lambda i: (0, 0)),
                  pl.BlockSpec((_DE, _D), lambda i: (0, 0)),
                  pl.BlockSpec((1, _D), lambda i: (0, 0))],
        out_specs=pl.BlockSpec((_N // 10, _D), lambda i: (i, 0)),
        out_shape=jax.ShapeDtypeStruct((_N, _D), jnp.float32),
    )(x, parts[0], parts[1], W_node[:_D], W_node[_D:], b_node.reshape(1, _D))

    return nodes, m


# speedup vs baseline: 1.0298x; 1.0298x over previous
import functools

import jax
import jax.numpy as jnp
from jax import lax
from jax.experimental import pallas as pl
from jax.experimental.pallas import tpu as pltpu
from jax.experimental.pallas import tpu_sc as plsc

_N = 10000
_E = 320000
_D = 128
_DE = 16
_EB = _E // 8

_NC = 2
_NS = 16
_NW = _NC * _NS
_EW = _E // _NW
_B = 1000
_NB = _EW // _B
_SB = 125
_JR = _B // _SB
_NP = 10240
_NPT = _NP // _NS


def _node_proj_body(x_ref, ws_ref, wr_ref, xs_ref, xr_ref):
    xv = x_ref[...]
    xs_ref[...] = jnp.dot(xv, ws_ref[...], preferred_element_type=jnp.float32)
    xr_ref[...] = jnp.dot(xv, wr_ref[...], preferred_element_type=jnp.float32)


def _edge_proj_body(ea_ref, bd_ref, b_ref, o_ref):
    o_ref[...] = jnp.dot(ea_ref[...], bd_ref[...],
                         preferred_element_type=jnp.float32) + b_ref[...]


def _node_mlp_body(x_ref, a0_ref, a1_ref, wt_ref, wb_ref, b_ref, o_ref):
    acc = jnp.dot(x_ref[...], wt_ref[...], preferred_element_type=jnp.float32)
    acc = acc + jnp.dot(a0_ref[...] + a1_ref[...], wb_ref[...],
                        preferred_element_type=jnp.float32)
    o_ref[...] = jnp.maximum(acc + b_ref[...], 0.0)


def _sc_edges_body(xs_hbm, xr_hbm, ea_hbm, s_hbm, r_hbm,
                   m_hbm, parts_hbm,
                   idx_s0, idx_r0, bufs0, bufr0, bufe0,
                   idx_s1, idx_r1, bufs1, bufr1, bufe1,
                   agg_sh,
                   sem_l0, sem_l1, sem_g0, sem_g1, sem_w):
    c = lax.axis_index("c")
    s = lax.axis_index("s")
    wid = c * _NS + s
    e0 = wid * _EW

    sets = ((idx_s0, idx_r0, bufs0, bufr0, bufe0, sem_l0, sem_g0),
            (idx_s1, idx_r1, bufs1, bufr1, bufe1, sem_l1, sem_g1))

    def _eb(b):
        return e0 + jnp.minimum(b, _NB - 1) * _B

    def _fire_loads(b, S):
        eb = _eb(b)
        pltpu.async_copy(s_hbm.at[pl.ds(eb, _B)], S[0], S[5])
        pltpu.async_copy(r_hbm.at[pl.ds(eb, _B)], S[1], S[5])
        pltpu.async_copy(ea_hbm.at[pl.ds(eb * _DE, _B * _DE)], S[4], S[5])

    def _drain_loads(S):
        eb = e0
        pltpu.make_async_copy(s_hbm.at[pl.ds(eb, _B)], S[0], S[5]).wait()
        pltpu.make_async_copy(r_hbm.at[pl.ds(eb, _B)], S[1], S[5]).wait()
        pltpu.make_async_copy(ea_hbm.at[pl.ds(eb * _DE, _B * _DE)], S[4],
                              S[5]).wait()

    def _fire_gathers(S):
        pltpu.async_copy(xs_hbm.at[S[0]], S[2], S[6])
        pltpu.async_copy(xr_hbm.at[S[1]], S[3], S[6])

    def _drain_gathers(S):
        pltpu.make_async_copy(xs_hbm.at[S[0]], S[2], S[6]).wait()
        pltpu.make_async_copy(xr_hbm.at[S[1]], S[3], S[6]).wait()

    def _zero(i, carry):
        bufs0[i] = jnp.zeros((_DE,), jnp.float32)
        return carry
    lax.fori_loop(0, _NPT, _zero, 0)
    pltpu.sync_copy(bufs0.at[pl.ds(0, _NPT)], agg_sh.at[pl.ds(s * _NPT, _NPT)])
    plsc.subcore_barrier()

    _fire_loads(0, sets[0])
    _drain_loads(sets[0])
    _fire_gathers(sets[0])
    _fire_loads(1, sets[1])

    def _process(b, S, T):
        _drain_gathers(S)

        bufs, bufr, bufe = S[2], S[3], S[4]

        def _relu(i2, carry):
            for u in range(8):
                row = i2 * 8 + u
                bufs[row] = jnp.maximum(
                    bufs[row] + bufr[row] + bufe[pl.ds(row * _DE, _DE)], 0.0)
            return carry
        lax.fori_loop(0, _B // 8, _relu, 0)

        eb = e0 + b * _B
        cpw = pltpu.async_copy(bufs, m_hbm.at[pl.ds(eb, _B)], sem_w)
        pltpu.sync_copy(bufs, agg_sh.at[S[1]], add=True)
        cpw.wait()

        _fire_loads(b + 2, S)
        _drain_loads(T)
        _fire_gathers(T)

    def _pair(i, carry):
        _process(2 * i, sets[0], sets[1])
        _process(2 * i + 1, sets[1], sets[0])
        return carry
    lax.fori_loop(0, _NB // 2, _pair, 0)

    _drain_loads(sets[1])
    _drain_gathers(sets[0])

    plsc.subcore_barrier()
    pltpu.sync_copy(agg_sh.at[pl.ds(s * _NPT, _NPT)],
                    bufs0.at[pl.ds(0, _NPT)])

    def _flatten(i, carry):
        bufe0[pl.ds(i * _DE, _DE)] = bufs0[i]
        return carry
    lax.fori_loop(0, _NPT, _flatten, 0)
    pltpu.sync_copy(bufe0.at[pl.ds(0, _NPT * _DE)],
                    parts_hbm.at[pl.ds((c * _NP + s * _NPT) * _DE,
                                       _NPT * _DE)])


_scratch_set = [
    pltpu.VMEM((_B,), jnp.int32),
    pltpu.VMEM((_B,), jnp.int32),
    pltpu.VMEM((_B, _DE), jnp.float32),
    pltpu.VMEM((_B, _DE), jnp.float32),
    pltpu.VMEM((_B * _DE,), jnp.float32),
]

_sc_edges = functools.partial(
    pl.kernel,
    out_type=(jax.ShapeDtypeStruct((_E, _DE), jnp.float32),
              jax.ShapeDtypeStruct((_NC * _NP * _DE,), jnp.float32)),
    mesh=plsc.VectorSubcoreMesh(core_axis_name="c", subcore_axis_name="s"),
    compiler_params=pltpu.CompilerParams(use_tc_tiling_on_sc=False),
    scratch_types=_scratch_set + _scratch_set + [
        pltpu.VMEM_SHARED((_NP, _DE), jnp.float32),
        pltpu.SemaphoreType.DMA,
        pltpu.SemaphoreType.DMA,
        pltpu.SemaphoreType.DMA,
        pltpu.SemaphoreType.DMA,
        pltpu.SemaphoreType.DMA,
    ],
)(_sc_edges_body)


@jax.jit
def kernel(x, edge_index, edge_attr, W_edge, b_edge, W_node, b_node):
    senders = edge_index[0]
    receivers = edge_index[1]
    w_s = W_edge[:_D]
    w_r = W_edge[_D:2 * _D]
    w_a = W_edge[2 * _D:]

    xs, xr = pl.pallas_call(
        _node_proj_body,
        out_shape=(jax.ShapeDtypeStruct((_N, _DE), jnp.float32),
                   jax.ShapeDtypeStruct((_N, _DE), jnp.float32)),
    )(x, w_s, w_r)

    ea2 = edge_attr.reshape(_EB, _D)
    bd = jnp.kron(jnp.eye(8, dtype=jnp.float32), w_a)
    bt = jnp.tile(b_edge, 8).reshape(1, _D)
    ea = pl.pallas_call(
        _edge_proj_body,
        grid=(10,),
        in_specs=[pl.BlockSpec((_EB // 10, _D), lambda i: (i, 0)),
                  pl.BlockSpec((_D, _D), lambda i: (0, 0)),
                  pl.BlockSpec((1, _D), lambda i: (0, 0))],
        out_specs=pl.BlockSpec((_EB // 10, _D), lambda i: (i, 0)),
        out_shape=jax.ShapeDtypeStruct((_EB, _D), jnp.float32),
    )(ea2, bd, bt)

    m, parts_flat = _sc_edges(xs, xr, ea.reshape(-1), senders, receivers)
    parts = parts_flat.reshape(_NC, _NP, _DE)[:, :_N]

    nodes = pl.pallas_call(
        _node_mlp_body,
        grid=(10,),
        in_specs=[pl.BlockSpec((_N // 10, _D), lambda i: (i, 0)),
                  pl.BlockSpec((_N // 10, _DE), lambda i: (i, 0)),
                  pl.BlockSpec((_N // 10, _DE), lambda i: (i, 0)),
                  pl.BlockSpec((_D, _D), lambda i: (0, 0)),
                  pl.BlockSpec((_DE, _D), lambda i: (0, 0)),
                  pl.BlockSpec((1, _D), lambda i: (0, 0))],
        out_specs=pl.BlockSpec((_N // 10, _D), lambda i: (i, 0)),
        out_shape=jax.ShapeDtypeStruct((_N, _D), jnp.float32),
    )(x, parts[0], parts[1], W_node[:_D], W_node[_D:], b_node.reshape(1, _D))

    return nodes, m


# speedup vs baseline: 1.6282x; 1.5811x over previous
import functools

import jax
import jax.numpy as jnp
from jax import lax
from jax.experimental import pallas as pl
from jax.experimental.pallas import tpu as pltpu
from jax.experimental.pallas import tpu_sc as plsc

_N = 10000
_E = 320000
_D = 128
_DE = 16
_EB = _E // 8

_NC = 2
_NS = 16
_NW = _NC * _NS
_EW = _E // _NW
_B = 1000
_NB = _EW // _B
_SB = 125
_JR = _B // _SB
_NP = 10240
_NPT = _NP // _NS


def _node_proj_body(x_ref, ws_ref, wr_ref, xs_ref, xr_ref):
    xv = x_ref[...]
    xs_ref[...] = jnp.dot(xv, ws_ref[...], preferred_element_type=jnp.float32)
    xr_ref[...] = jnp.dot(xv, wr_ref[...], preferred_element_type=jnp.float32)


def _edge_proj_body(eaT_ref, waT_ref, b_ref, o_ref):
    o_ref[...] = jnp.dot(waT_ref[...], eaT_ref[...],
                         preferred_element_type=jnp.float32) + b_ref[...]


def _node_mlp_body(x_ref, a0_ref, a1_ref, wt_ref, wb_ref, b_ref, o_ref):
    acc = jnp.dot(x_ref[...], wt_ref[...], preferred_element_type=jnp.float32)
    acc = acc + jnp.dot(a0_ref[...] + a1_ref[...], wb_ref[...],
                        preferred_element_type=jnp.float32)
    o_ref[...] = jnp.maximum(acc + b_ref[...], 0.0)


def _sc_edges_body(xs_hbm, xr_hbm, eaT_hbm, s_hbm, r_hbm,
                   mT_hbm, parts_hbm,
                   idx_s0, idx_r0, bufs0, bufe0,
                   idx_s1, idx_r1, bufs1, bufe1,
                   bufmT, stage, agg_sh,
                   sem_l0, sem_l1, sem_g0, sem_g1, sem_w):
    c = lax.axis_index("c")
    s = lax.axis_index("s")
    wid = c * _NS + s
    e0 = wid * _EW
    lanes = lax.iota(jnp.int32, _DE)

    sets = ((idx_s0, idx_r0, bufs0, bufe0, sem_l0, sem_g0),
            (idx_s1, idx_r1, bufs1, bufe1, sem_l1, sem_g1))

    def _eb(b):
        return e0 + jnp.minimum(b, _NB - 1) * _B

    def _fire_loads(b, S):
        eb = _eb(b)
        pltpu.async_copy(s_hbm.at[pl.ds(eb, _B)], S[0], S[4])
        pltpu.async_copy(r_hbm.at[pl.ds(eb, _B)], S[1], S[4])
        pltpu.async_copy(eaT_hbm.at[:, pl.ds(eb, _B)], S[3], S[4])

    def _drain_loads(S):
        eb = e0
        pltpu.make_async_copy(s_hbm.at[pl.ds(eb, _B)], S[0], S[4]).wait()
        pltpu.make_async_copy(r_hbm.at[pl.ds(eb, _B)], S[1], S[4]).wait()
        pltpu.make_async_copy(eaT_hbm.at[:, pl.ds(eb, _B)], S[3], S[4]).wait()

    def _fire_gather_xs(S):
        pltpu.async_copy(xs_hbm.at[S[0]], S[2], S[5])

    def _fire_gather_xr_add(S):
        pltpu.async_copy(xr_hbm.at[S[1]], S[2], S[5], add=True)

    def _drain_gather(S):
        pltpu.make_async_copy(xs_hbm.at[S[0]], S[2], S[5]).wait()

    def _zero(i, carry):
        bufs0[i] = jnp.zeros((_DE,), jnp.float32)
        return carry
    lax.fori_loop(0, _NPT, _zero, 0)
    pltpu.sync_copy(bufs0.at[pl.ds(0, _NPT)], agg_sh.at[pl.ds(s * _NPT, _NPT)])
    plsc.subcore_barrier()

    _fire_loads(0, sets[0])
    _drain_loads(sets[0])
    _fire_gather_xs(sets[0])
    _fire_loads(1, sets[1])
    _drain_gather(sets[0])
    _fire_gather_xr_add(sets[0])

    def _process(b, S, T):
        _drain_loads(T)
        _fire_gather_xs(T)
        _drain_gather(S)

        bufs, bufe = S[2], S[3]

        def _relu(i2, carry):
            for u in range(8):
                row = i2 * 8 + u
                col = jnp.full((_DE,), row, jnp.int32)
                eac = plsc.load_gather(bufe, [lanes, col])
                val = jnp.maximum(bufs[row] + eac, 0.0)
                bufs[row] = val
                plsc.store_scatter(bufmT, [lanes, col], val)
            return carry
        lax.fori_loop(0, _B // 8, _relu, 0)

        eb = e0 + b * _B
        cpw = pltpu.async_copy(bufmT, mT_hbm.at[:, pl.ds(eb, _B)], sem_w)
        pltpu.sync_copy(bufs, agg_sh.at[S[1]], add=True)
        cpw.wait()

        _fire_loads(b + 2, S)
        _drain_gather(T)
        _fire_gather_xr_add(T)

    def _pair(i, carry):
        _process(2 * i, sets[0], sets[1])
        _process(2 * i + 1, sets[1], sets[0])
        return carry
    lax.fori_loop(0, _NB // 2, _pair, 0)

    _drain_loads(sets[1])
    _drain_gather(sets[0])

    plsc.subcore_barrier()
    pltpu.sync_copy(agg_sh.at[pl.ds(s * _NPT, _NPT)],
                    bufs0.at[pl.ds(0, _NPT)])

    def _flatten(i, carry):
        stage[pl.ds(i * _DE, _DE)] = bufs0[i]
        return carry
    lax.fori_loop(0, _NPT, _flatten, 0)
    pltpu.sync_copy(stage,
                    parts_hbm.at[pl.ds((c * _NP + s * _NPT) * _DE,
                                       _NPT * _DE)])


_scratch_set = [
    pltpu.VMEM((_B,), jnp.int32),
    pltpu.VMEM((_B,), jnp.int32),
    pltpu.VMEM((_B, _DE), jnp.float32),
    pltpu.VMEM((_DE, _B), jnp.float32),
]

_sc_edges = functools.partial(
    pl.kernel,
    out_type=(jax.ShapeDtypeStruct((_DE, _E), jnp.float32),
              jax.ShapeDtypeStruct((_NC * _NP * _DE,), jnp.float32)),
    mesh=plsc.VectorSubcoreMesh(core_axis_name="c", subcore_axis_name="s"),
    compiler_params=pltpu.CompilerParams(use_tc_tiling_on_sc=False,
                                         needs_layout_passes=False),
    scratch_types=_scratch_set + _scratch_set + [
        pltpu.VMEM((_DE, _B), jnp.float32),
        pltpu.VMEM((_NPT * _DE,), jnp.float32),
        pltpu.VMEM_SHARED((_NP, _DE), jnp.float32),
        pltpu.SemaphoreType.DMA,
        pltpu.SemaphoreType.DMA,
        pltpu.SemaphoreType.DMA,
        pltpu.SemaphoreType.DMA,
        pltpu.SemaphoreType.DMA,
    ],
)(_sc_edges_body)


@jax.jit
def kernel(x, edge_index, edge_attr, W_edge, b_edge, W_node, b_node):
    senders = edge_index[0]
    receivers = edge_index[1]
    w_s = W_edge[:_D]
    w_r = W_edge[_D:2 * _D]
    w_a = W_edge[2 * _D:]

    xs, xr = pl.pallas_call(
        _node_proj_body,
        out_shape=(jax.ShapeDtypeStruct((_N, _DE), jnp.float32),
                   jax.ShapeDtypeStruct((_N, _DE), jnp.float32)),
    )(x, w_s, w_r)

    eaT = edge_attr.T
    ea = pl.pallas_call(
        _edge_proj_body,
        grid=(10,),
        in_specs=[pl.BlockSpec((_DE, _E // 10), lambda i: (0, i)),
                  pl.BlockSpec((_DE, _DE), lambda i: (0, 0)),
                  pl.BlockSpec((_DE, 1), lambda i: (0, 0))],
        out_specs=pl.BlockSpec((_DE, _E // 10), lambda i: (0, i)),
        out_shape=jax.ShapeDtypeStruct((_DE, _E), jnp.float32),
    )(eaT, w_a.T, b_edge.reshape(_DE, 1))

    mT, parts_flat = _sc_edges(xs, xr, ea, senders, receivers)
    m = mT.T
    parts = parts_flat.reshape(_NC, _NP, _DE)[:, :_N]

    nodes = pl.pallas_call(
        _node_mlp_body,
        grid=(10,),
        in_specs=[pl.BlockSpec((_N // 10, _D), lambda i: (i, 0)),
                  pl.BlockSpec((_N // 10, _DE), lambda i: (i, 0)),
                  pl.BlockSpec((_N // 10, _DE), lambda i: (i, 0)),
                  pl.BlockSpec((_D, _D), lambda i: (0, 0)),
                  pl.BlockSpec((_DE, _D), lambda i: (0, 0)),
                  pl.BlockSpec((1, _D), lambda i: (0, 0))],
        out_specs=pl.BlockSpec((_N // 10, _D), lambda i: (i, 0)),
        out_shape=jax.ShapeDtypeStruct((_N, _D), jnp.float32),
    )(x, parts[0], parts[1], W_node[:_D], W_node[_D:], b_node.reshape(1, _D))

    return nodes, m


# speedup vs baseline: 1.7935x; 1.1016x over previous
import functools

import jax
import jax.numpy as jnp
from jax import lax
from jax.experimental import pallas as pl
from jax.experimental.pallas import tpu as pltpu
from jax.experimental.pallas import tpu_sc as plsc

_N = 10000
_E = 320000
_D = 128
_DE = 16
_EB = _E // 8

_NC = 2
_NS = 16
_NW = _NC * _NS
_EW = _E // _NW
_B = 1000
_NB = _EW // _B
_SB = 125
_JR = _B // _SB
_NP = 10240
_NPT = _NP // _NS


def _node_proj_body(x_ref, ws_ref, wr_ref, xs_ref, xr_ref):
    xv = x_ref[...]
    xs_ref[...] = jnp.dot(xv, ws_ref[...], preferred_element_type=jnp.float32)
    xr_ref[...] = jnp.dot(xv, wr_ref[...], preferred_element_type=jnp.float32)


def _edge_proj_body(eaT_ref, waT_ref, b_ref, o_ref):
    o_ref[...] = jnp.dot(waT_ref[...], eaT_ref[...],
                         preferred_element_type=jnp.float32) + b_ref[...]


def _node_mlp_body(x_ref, a0_ref, a1_ref, wt_ref, wb_ref, b_ref, o_ref):
    acc = jnp.dot(x_ref[...], wt_ref[...], preferred_element_type=jnp.float32)
    aT = a0_ref[...] + a1_ref[...]
    acc = acc + lax.dot_general(aT, wb_ref[...], (((0,), (0,)), ((), ())),
                                preferred_element_type=jnp.float32)
    o_ref[...] = jnp.maximum(acc + b_ref[...], 0.0)


def _sc_edges_body(xs_hbm, xr_hbm, eaT_hbm, ei_hbm,
                   mT_hbm, parts_hbm,
                   idx_s0, idx_r0, bufs0, bufe0,
                   idx_s1, idx_r1, bufs1, bufe1,
                   bufmT, stage, agg_sh,
                   sem_l0, sem_l1, sem_g0, sem_g1, sem_w):
    c = lax.axis_index("c")
    s = lax.axis_index("s")
    wid = c * _NS + s
    e0 = wid * _EW
    lanes = lax.iota(jnp.int32, _DE)

    sets = ((idx_s0, idx_r0, bufs0, bufe0, sem_l0, sem_g0),
            (idx_s1, idx_r1, bufs1, bufe1, sem_l1, sem_g1))

    def _eb(b):
        return e0 + jnp.minimum(b, _NB - 1) * _B

    def _fire_loads(b, S):
        eb = _eb(b)
        pltpu.async_copy(ei_hbm.at[0, pl.ds(eb, _B)], S[0], S[4])
        pltpu.async_copy(ei_hbm.at[1, pl.ds(eb, _B)], S[1], S[4])
        pltpu.async_copy(eaT_hbm.at[:, pl.ds(eb, _B)], S[3], S[4])

    def _drain_loads(S):
        eb = e0
        pltpu.make_async_copy(ei_hbm.at[0, pl.ds(eb, _B)], S[0], S[4]).wait()
        pltpu.make_async_copy(ei_hbm.at[1, pl.ds(eb, _B)], S[1], S[4]).wait()
        pltpu.make_async_copy(eaT_hbm.at[:, pl.ds(eb, _B)], S[3], S[4]).wait()

    def _fire_gather_xs(S):
        pltpu.async_copy(xs_hbm.at[S[0]], S[2], S[5])

    def _fire_gather_xr_add(S):
        pltpu.async_copy(xr_hbm.at[S[1]], S[2], S[5], add=True)

    def _drain_gather(S):
        pltpu.make_async_copy(xs_hbm.at[S[0]], S[2], S[5]).wait()

    def _zero(i, carry):
        bufs0[i] = jnp.zeros((_DE,), jnp.float32)
        return carry
    lax.fori_loop(0, _NPT, _zero, 0)
    pltpu.sync_copy(bufs0.at[pl.ds(0, _NPT)], agg_sh.at[pl.ds(s * _NPT, _NPT)])
    plsc.subcore_barrier()

    _fire_loads(0, sets[0])
    _drain_loads(sets[0])
    _fire_gather_xs(sets[0])
    _fire_loads(1, sets[1])
    _drain_gather(sets[0])
    _fire_gather_xr_add(sets[0])

    def _process(b, S, T):
        _drain_loads(T)
        _fire_gather_xs(T)
        _drain_gather(S)

        bufs, bufe = S[2], S[3]

        def _relu(i2, carry):
            for u in range(8):
                row = i2 * 8 + u
                col = jnp.full((_DE,), row, jnp.int32)
                eac = plsc.load_gather(bufe, [lanes, col])
                val = jnp.maximum(bufs[row] + eac, 0.0)
                bufs[row] = val
                plsc.store_scatter(bufmT, [lanes, col], val)
            return carry
        lax.fori_loop(0, _B // 8, _relu, 0)

        _drain_gather(T)
        _fire_gather_xr_add(T)

        eb = e0 + b * _B
        cpw = pltpu.async_copy(bufmT, mT_hbm.at[:, pl.ds(eb, _B)], sem_w)
        pltpu.sync_copy(bufs, agg_sh.at[S[1]], add=True)
        cpw.wait()

        _fire_loads(b + 2, S)

    def _pair(i, carry):
        _process(2 * i, sets[0], sets[1])
        _process(2 * i + 1, sets[1], sets[0])
        return carry
    lax.fori_loop(0, _NB // 2, _pair, 0)

    _drain_loads(sets[1])
    _drain_gather(sets[0])

    plsc.subcore_barrier()
    pltpu.sync_copy(agg_sh.at[pl.ds(s * _NPT, _NPT)],
                    bufs0.at[pl.ds(0, _NPT)])

    def _flatten(i, carry):
        col = jnp.full((_DE,), i, jnp.int32)
        plsc.store_scatter(stage, [lanes, col], bufs0[i])
        return carry
    lax.fori_loop(0, _NPT, _flatten, 0)
    pltpu.sync_copy(stage,
                    parts_hbm.at[:, pl.ds(c * _NP + s * _NPT, _NPT)])


_scratch_set = [
    pltpu.VMEM((_B,), jnp.int32),
    pltpu.VMEM((_B,), jnp.int32),
    pltpu.VMEM((_B, _DE), jnp.float32),
    pltpu.VMEM((_DE, _B), jnp.float32),
]

_sc_edges = functools.partial(
    pl.kernel,
    out_type=(jax.ShapeDtypeStruct((_DE, _E), jnp.float32),
              jax.ShapeDtypeStruct((_DE, _NC * _NP), jnp.float32)),
    mesh=plsc.VectorSubcoreMesh(core_axis_name="c", subcore_axis_name="s"),
    compiler_params=pltpu.CompilerParams(use_tc_tiling_on_sc=False,
                                         needs_layout_passes=False),
    scratch_types=_scratch_set + _scratch_set + [
        pltpu.VMEM((_DE, _B), jnp.float32),
        pltpu.VMEM((_DE, _NPT), jnp.float32),
        pltpu.VMEM_SHARED((_NP, _DE), jnp.float32),
        pltpu.SemaphoreType.DMA,
        pltpu.SemaphoreType.DMA,
        pltpu.SemaphoreType.DMA,
        pltpu.SemaphoreType.DMA,
        pltpu.SemaphoreType.DMA,
    ],
)(_sc_edges_body)


@jax.jit
def kernel(x, edge_index, edge_attr, W_edge, b_edge, W_node, b_node):
    w_s = W_edge[:_D]
    w_r = W_edge[_D:2 * _D]
    w_a = W_edge[2 * _D:]

    xs, xr = pl.pallas_call(
        _node_proj_body,
        out_shape=(jax.ShapeDtypeStruct((_N, _DE), jnp.float32),
                   jax.ShapeDtypeStruct((_N, _DE), jnp.float32)),
    )(x, w_s, w_r)

    eaT = edge_attr.T
    ea = pl.pallas_call(
        _edge_proj_body,
        grid=(10,),
        in_specs=[pl.BlockSpec((_DE, _E // 10), lambda i: (0, i)),
                  pl.BlockSpec((_DE, _DE), lambda i: (0, 0)),
                  pl.BlockSpec((_DE, 1), lambda i: (0, 0))],
        out_specs=pl.BlockSpec((_DE, _E // 10), lambda i: (0, i)),
        out_shape=jax.ShapeDtypeStruct((_DE, _E), jnp.float32),
    )(eaT, w_a.T, b_edge.reshape(_DE, 1))

    mT, partsT = _sc_edges(xs, xr, ea, edge_index)
    m = mT.T

    nodes = pl.pallas_call(
        _node_mlp_body,
        grid=(10,),
        in_specs=[pl.BlockSpec((1024, _D), lambda i: (i, 0)),
                  pl.BlockSpec((_DE, 1024), lambda i: (0, i)),
                  pl.BlockSpec((_DE, 1024), lambda i: (0, i)),
                  pl.BlockSpec((_D, _D), lambda i: (0, 0)),
                  pl.BlockSpec((_DE, _D), lambda i: (0, 0)),
                  pl.BlockSpec((1, _D), lambda i: (0, 0))],
        out_specs=pl.BlockSpec((1024, _D), lambda i: (i, 0)),
        out_shape=jax.ShapeDtypeStruct((_N, _D), jnp.float32),
    )(x, partsT[:, :_NP], partsT[:, _NP:], W_node[:_D], W_node[_D:],
      b_node.reshape(1, _D))

    return nodes, m


# speedup vs baseline: 1.8257x; 1.0180x over previous
import functools

import jax
import jax.numpy as jnp
from jax import lax
from jax.experimental import pallas as pl
from jax.experimental.pallas import tpu as pltpu
from jax.experimental.pallas import tpu_sc as plsc

_N = 10000
_E = 320000
_D = 128
_DE = 16
_EB = _E // 8

_NC = 2
_NS = 16
_NW = _NC * _NS
_EW = _E // _NW
_B = 1000
_NB = _EW // _B
_SB = 125
_JR = _B // _SB
_NP = 10240
_NPT = _NP // _NS


def _node_proj_body(x_ref, ws_ref, wr_ref, xs_ref, xr_ref):
    xv = x_ref[...]
    xs_ref[...] = jnp.dot(xv, ws_ref[...], preferred_element_type=jnp.float32)
    xr_ref[...] = jnp.dot(xv, wr_ref[...], preferred_element_type=jnp.float32)


def _edge_proj_body(eaT_ref, waT_ref, b_ref, o_ref):
    o_ref[...] = jnp.dot(waT_ref[...], eaT_ref[...],
                         preferred_element_type=jnp.float32) + b_ref[...]


def _node_mlp_body(x_ref, a0_ref, a1_ref, wt_ref, wb_ref, b_ref, o_ref):
    acc = jnp.dot(x_ref[...], wt_ref[...], preferred_element_type=jnp.float32)
    aT = a0_ref[...] + a1_ref[...]
    acc = acc + lax.dot_general(aT, wb_ref[...], (((0,), (0,)), ((), ())),
                                preferred_element_type=jnp.float32)
    o_ref[...] = jnp.maximum(acc + b_ref[...], 0.0)


def _sc_edges_body(xs_hbm, xr_hbm, eaT_hbm, ei_hbm,
                   mT_hbm, parts_hbm,
                   idx_s0, idx_r0, bufs0, bufe0,
                   idx_s1, idx_r1, bufs1, bufe1,
                   bufmT0, bufmT1, agg_sh,
                   sem_l0, sem_l1, sem_g0, sem_g1, sem_w0, sem_w1):
    c = lax.axis_index("c")
    s = lax.axis_index("s")
    wid = c * _NS + s
    e0 = wid * _EW
    lanes = lax.iota(jnp.int32, _DE)

    sets = ((idx_s0, idx_r0, bufs0, bufe0, sem_l0, sem_g0, bufmT0, sem_w0),
            (idx_s1, idx_r1, bufs1, bufe1, sem_l1, sem_g1, bufmT1, sem_w1))

    def _eb(b):
        return e0 + jnp.minimum(b, _NB - 1) * _B

    def _fire_loads(b, S):
        eb = _eb(b)
        pltpu.async_copy(ei_hbm.at[0, pl.ds(eb, _B)], S[0], S[4])
        pltpu.async_copy(ei_hbm.at[1, pl.ds(eb, _B)], S[1], S[4])
        pltpu.async_copy(eaT_hbm.at[:, pl.ds(eb, _B)], S[3], S[4])

    def _drain_loads(S):
        eb = e0
        pltpu.make_async_copy(ei_hbm.at[0, pl.ds(eb, _B)], S[0], S[4]).wait()
        pltpu.make_async_copy(ei_hbm.at[1, pl.ds(eb, _B)], S[1], S[4]).wait()
        pltpu.make_async_copy(eaT_hbm.at[:, pl.ds(eb, _B)], S[3], S[4]).wait()

    def _fire_gather_xs(S):
        pltpu.async_copy(xs_hbm.at[S[0]], S[2], S[5])

    def _fire_gather_xr_add(S):
        pltpu.async_copy(xr_hbm.at[S[1]], S[2], S[5], add=True)

    def _drain_gather(S):
        pltpu.make_async_copy(xs_hbm.at[S[0]], S[2], S[5]).wait()

    def _zero(i, carry):
        bufs0[i] = jnp.zeros((_DE,), jnp.float32)
        return carry
    lax.fori_loop(0, _NPT, _zero, 0)
    pltpu.sync_copy(bufs0.at[pl.ds(0, _NPT)], agg_sh.at[pl.ds(s * _NPT, _NPT)])
    plsc.subcore_barrier()

    _fire_loads(0, sets[0])
    _drain_loads(sets[0])
    _fire_gather_xs(sets[0])
    _fire_loads(1, sets[1])
    _drain_gather(sets[0])
    _fire_gather_xr_add(sets[0])
    lanes0 = lanes

    def _drain_mwrite(S):
        pltpu.make_async_copy(S[6], mT_hbm.at[:, pl.ds(e0, _B)],
                              S[7]).wait()

    def _process(b, S, T, warm):
        _drain_loads(T)
        _fire_gather_xs(T)
        _drain_gather(S)

        bufs, bufe = S[2], S[3]
        bufmT = S[6]

        @pl.when(warm)
        def _():
            _drain_mwrite(S)

        def _relu(i2, carry):
            for u in range(8):
                row = i2 * 8 + u
                col = jnp.full((_DE,), row, jnp.int32)
                eac = plsc.load_gather(bufe, [lanes, col])
                val = jnp.maximum(bufs[row] + eac, 0.0)
                bufs[row] = val
                plsc.store_scatter(bufmT, [lanes, col], val)
            return carry
        lax.fori_loop(0, _B // 8, _relu, 0)

        _drain_gather(T)
        _fire_gather_xr_add(T)

        eb = e0 + b * _B
        pltpu.async_copy(bufmT, mT_hbm.at[:, pl.ds(eb, _B)], S[7])
        pltpu.sync_copy(bufs, agg_sh.at[S[1]], add=True)

        _fire_loads(b + 2, S)

    def _pair(i, carry):
        warm = i >= 1
        _process(2 * i, sets[0], sets[1], warm)
        _process(2 * i + 1, sets[1], sets[0], warm)
        return carry
    lax.fori_loop(0, _NB // 2, _pair, 0)

    _drain_loads(sets[1])
    _drain_gather(sets[0])
    _drain_mwrite(sets[0])
    _drain_mwrite(sets[1])

    plsc.subcore_barrier()
    pltpu.sync_copy(agg_sh.at[pl.ds(s * _NPT, _NPT)],
                    bufs0.at[pl.ds(0, _NPT)])

    def _flatten(i, carry):
        col = jnp.full((_DE,), i, jnp.int32)
        plsc.store_scatter(bufmT0, [lanes0, col], bufs0[i])
        return carry
    lax.fori_loop(0, _NPT, _flatten, 0)
    pltpu.sync_copy(bufmT0.at[:, pl.ds(0, _NPT)],
                    parts_hbm.at[:, pl.ds(c * _NP + s * _NPT, _NPT)])


_scratch_set = [
    pltpu.VMEM((_B,), jnp.int32),
    pltpu.VMEM((_B,), jnp.int32),
    pltpu.VMEM((_B, _DE), jnp.float32),
    pltpu.VMEM((_DE, _B), jnp.float32),
]

_sc_edges = functools.partial(
    pl.kernel,
    out_type=(jax.ShapeDtypeStruct((_DE, _E), jnp.float32),
              jax.ShapeDtypeStruct((_DE, _NC * _NP), jnp.float32)),
    mesh=plsc.VectorSubcoreMesh(core_axis_name="c", subcore_axis_name="s"),
    compiler_params=pltpu.CompilerParams(use_tc_tiling_on_sc=False,
                                         needs_layout_passes=False),
    scratch_types=_scratch_set + _scratch_set + [
        pltpu.VMEM((_DE, _B), jnp.float32),
        pltpu.VMEM((_DE, _B), jnp.float32),
        pltpu.VMEM_SHARED((_NP, _DE), jnp.float32),
        pltpu.SemaphoreType.DMA,
        pltpu.SemaphoreType.DMA,
        pltpu.SemaphoreType.DMA,
        pltpu.SemaphoreType.DMA,
        pltpu.SemaphoreType.DMA,
        pltpu.SemaphoreType.DMA,
    ],
)(_sc_edges_body)


@jax.jit
def kernel(x, edge_index, edge_attr, W_edge, b_edge, W_node, b_node):
    w_s = W_edge[:_D]
    w_r = W_edge[_D:2 * _D]
    w_a = W_edge[2 * _D:]

    xs, xr = pl.pallas_call(
        _node_proj_body,
        out_shape=(jax.ShapeDtypeStruct((_N, _DE), jnp.float32),
                   jax.ShapeDtypeStruct((_N, _DE), jnp.float32)),
    )(x, w_s, w_r)

    eaT = edge_attr.T
    ea = pl.pallas_call(
        _edge_proj_body,
        grid=(10,),
        in_specs=[pl.BlockSpec((_DE, _E // 10), lambda i: (0, i)),
                  pl.BlockSpec((_DE, _DE), lambda i: (0, 0)),
                  pl.BlockSpec((_DE, 1), lambda i: (0, 0))],
        out_specs=pl.BlockSpec((_DE, _E // 10), lambda i: (0, i)),
        out_shape=jax.ShapeDtypeStruct((_DE, _E), jnp.float32),
    )(eaT, w_a.T, b_edge.reshape(_DE, 1))

    mT, partsT = _sc_edges(xs, xr, ea, edge_index)
    m = mT.T

    nodes = pl.pallas_call(
        _node_mlp_body,
        grid=(10,),
        in_specs=[pl.BlockSpec((1024, _D), lambda i: (i, 0)),
                  pl.BlockSpec((_DE, 1024), lambda i: (0, i)),
                  pl.BlockSpec((_DE, 1024), lambda i: (0, i)),
                  pl.BlockSpec((_D, _D), lambda i: (0, 0)),
                  pl.BlockSpec((_DE, _D), lambda i: (0, 0)),
                  pl.BlockSpec((1, _D), lambda i: (0, 0))],
        out_specs=pl.BlockSpec((1024, _D), lambda i: (i, 0)),
        out_shape=jax.ShapeDtypeStruct((_N, _D), jnp.float32),
    )(x, partsT[:, :_NP], partsT[:, _NP:], W_node[:_D], W_node[_D:],
      b_node.reshape(1, _D))

    return nodes, m
